# baseline (speedup 1.0000x reference)
import functools

import jax
import jax.numpy as jnp
from jax.experimental import pallas as pl
from jax.experimental.pallas import tpu as pltpu


def _se_kernel(x_ref, w1_hbm, w2_hbm, o_ref, w1_v, w2_v, sem1, sem2,
               *, inv_hw):
    cp1 = pltpu.make_async_copy(w1_hbm, w1_v, sem1)
    cp1.start()
    cp2 = pltpu.make_async_copy(w2_hbm, w2_v, sem2)
    cp2.start()
    x = x_ref[...]
    pooled = jnp.sum(x, axis=0) * inv_hw
    cp1.wait()
    cp2.wait()
    h = jax.lax.dot_general(
        pooled, w1_v[...], (((1,), (1,)), ((), ())),
        preferred_element_type=jnp.float32)
    h = jnp.maximum(h, 0.0)
    g = jax.lax.dot_general(
        h, w2_v[...], (((1,), (1,)), ((), ())),
        preferred_element_type=jnp.float32)
    g = jax.nn.sigmoid(g)
    o_ref[...] = x * g[None, :, :]


def kernel(x, w1, w2):
    b, c, h, w = x.shape
    hw = h * w
    c_red = w1.shape[0]
    itemsize = jnp.dtype(x.dtype).itemsize

    xt = jnp.transpose(x.reshape(b, c, hw), (2, 0, 1))

    tb = 16
    while b % tb:
        tb -= 1

    w_bytes = int((w1.size + w2.size) * jnp.dtype(w1.dtype).itemsize)
    cost = pl.CostEstimate(
        flops=int(2 * b * c * hw + 4 * b * c * c_red),
        transcendentals=int(b * c),
        bytes_accessed=int(2 * b * c * hw * itemsize + w_bytes))

    out_t = pl.pallas_call(
        functools.partial(_se_kernel, inv_hw=1.0 / hw),
        out_shape=jax.ShapeDtypeStruct((hw, b, c), x.dtype),
        grid=(b // tb,),
        in_specs=[
            pl.BlockSpec((hw, tb, c), lambda i: (0, i, 0)),
            pl.BlockSpec(memory_space=pl.ANY),
            pl.BlockSpec(memory_space=pl.ANY),
        ],
        out_specs=pl.BlockSpec((hw, tb, c), lambda i: (0, i, 0)),
        scratch_shapes=[
            pltpu.VMEM((c_red, c), jnp.float32),
            pltpu.VMEM((c, c_red), jnp.float32),
            pltpu.SemaphoreType.DMA,
            pltpu.SemaphoreType.DMA,
        ],
        compiler_params=pltpu.CompilerParams(
            dimension_semantics=("parallel",),
            vmem_limit_bytes=48 * 1024 * 1024),
        cost_estimate=cost,
    )(xt, w1, w2)

    return jnp.transpose(out_t, (1, 2, 0)).reshape(b, c, h, w)
